# baseline (speedup 1.0000x reference)
import functools

import jax
import jax.numpy as jnp
from jax import lax
from jax.experimental import pallas as pl
from jax.experimental.pallas import tpu as pltpu
from jax.experimental.pallas import tpu_sc as plsc

BATCH = 16384
FEAT = 32
NC = 2
NS = 16
NW = NC * NS
BPW = BATCH // NW
BLK = 128
NBLK = BPW // BLK
LANES = 16
GROUPS_PER_BLK = BLK // LANES


def _body(idx_hbm, cb0_hbm, cb1_hbm, cb2_hbm, cb3_hbm, out_hbm,
          idx_v, g0, g1, g2, g3, b0, b1, b2, b3, acc, sem):
    wid = lax.axis_index("s") * NC + lax.axis_index("c")
    base = wid * BPW

    pltpu.sync_copy(idx_hbm.at[pl.ds(base, BPW)], idx_v)

    for b in range(NBLK):
        def mask_body(j, _, b=b):
            s = pl.ds(j * LANES, LANES)
            t = lax.shift_right_logical(idx_v[pl.ds(b * BLK + j * LANES, LANES)], 2)
            g0[b, s] = lax.bitwise_and(t, 1023)
            g1[b, s] = lax.bitwise_and(t, 4095)
            g2[b, s] = lax.bitwise_and(t, 16383)
            g3[b, s] = t
            return 0

        lax.fori_loop(0, GROUPS_PER_BLK, mask_body, 0, unroll=2)

    lane_iota = lax.iota(jnp.int32, LANES)
    dst_col_base = lax.mul(lax.bitwise_and(lane_iota, 3), 32)

    for b in range(NBLK):
        c0 = pltpu.async_copy(cb0_hbm.at[g0.at[b]], b0, sem)
        c1 = pltpu.async_copy(cb1_hbm.at[g1.at[b]], b1, sem)
        c2 = pltpu.async_copy(cb2_hbm.at[g2.at[b]], b2, sem)
        c3 = pltpu.async_copy(cb3_hbm.at[g3.at[b]], b3, sem)
        c0.wait()
        c1.wait()
        c2.wait()
        c3.wait()

        def group_body(g, _, b=b):
            q0 = b * BLK + g * LANES
            v = idx_v[pl.ds(q0, LANES)]
            src_col0 = lax.mul(lax.bitwise_and(v, 3), 32)
            src_row = lax.add(lane_iota, g * LANES)
            dst_row = lax.shift_right_logical(
                lax.add(lane_iota, q0), 2)
            for c in range(FEAT):
                col = lax.add(src_col0, c)
                f0 = plsc.load_gather(b0, [src_row, col])
                f1 = plsc.load_gather(b1, [src_row, col])
                f2 = plsc.load_gather(b2, [src_row, col])
                f3 = plsc.load_gather(b3, [src_row, col])
                tot = (f0 + f1) + (f2 + f3)
                plsc.store_scatter(acc, [dst_row, lax.add(dst_col_base, c)], tot)
            return 0

        lax.fori_loop(0, GROUPS_PER_BLK, group_body, 0)

    pltpu.sync_copy(acc, out_hbm.at[pl.ds(wid * BLK, BLK)])


@jax.jit
def _octree_lookup(indices, cb0, cb1, cb2, cb3):
    cbs = [jnp.reshape(cb, (-1, 128)) for cb in (cb0, cb1, cb2, cb3)]
    mesh = plsc.VectorSubcoreMesh(core_axis_name="c", subcore_axis_name="s")
    f = functools.partial(
        pl.kernel,
        mesh=mesh,
        compiler_params=pltpu.CompilerParams(needs_layout_passes=False),
        out_type=jax.ShapeDtypeStruct((BATCH // 4, 128), jnp.float32),
        scratch_types=[
            pltpu.VMEM((BPW,), jnp.int32),
            pltpu.VMEM((NBLK, BLK), jnp.int32),
            pltpu.VMEM((NBLK, BLK), jnp.int32),
            pltpu.VMEM((NBLK, BLK), jnp.int32),
            pltpu.VMEM((NBLK, BLK), jnp.int32),
            pltpu.VMEM((BLK, 128), jnp.float32),
            pltpu.VMEM((BLK, 128), jnp.float32),
            pltpu.VMEM((BLK, 128), jnp.float32),
            pltpu.VMEM((BLK, 128), jnp.float32),
            pltpu.VMEM((BLK, 128), jnp.float32),
            pltpu.SemaphoreType.DMA,
        ],
    )(_body)
    out = f(indices, *cbs)
    return jnp.reshape(out, (BATCH, FEAT))


def kernel(indices, cb0, cb1, cb2, cb3):
    return _octree_lookup(indices.astype(jnp.int32), cb0, cb1, cb2, cb3)


# speedup vs baseline: 2.3475x; 2.3475x over previous
import functools

import jax
import jax.numpy as jnp
from jax import lax
from jax.experimental import pallas as pl
from jax.experimental.pallas import tpu as pltpu
from jax.experimental.pallas import tpu_sc as plsc

BATCH = 16384
FEAT = 32
NC = 2
NS = 16
LANES = 16
NGROUPS = BATCH // LANES
CHUNK = 65536


def _body(idx_hbm, t0_hbm, t1_hbm, t2_hbm, t3_hbm, out_hbm,
          idx_v, acc, p0, p1, pbuf, sem):
    c = lax.axis_index("s") * NC + lax.axis_index("c")

    ci = pltpu.async_copy(idx_hbm, idx_v, sem)
    c0 = pltpu.async_copy(t0_hbm.at[c], p0, sem)
    c1 = pltpu.async_copy(t1_hbm.at[c], p1, sem)
    c2 = pltpu.async_copy(t2_hbm.at[c], pbuf, sem)
    ci.wait()
    c0.wait()
    c1.wait()
    c2.wait()

    def phase_a(j, _):
        s = pl.ds(j * LANES, LANES)
        v = idx_v[s]
        a = plsc.load_gather(p0, [lax.bitwise_and(v, 4095)])
        b = plsc.load_gather(p1, [lax.bitwise_and(v, 16383)])
        d = plsc.load_gather(pbuf, [lax.bitwise_and(v, 65535)])
        acc[s] = (a + b) + d
        return 0

    lax.fori_loop(0, NGROUPS, phase_a, 0, unroll=4)

    for k in range(4):
        ck = pltpu.async_copy(t3_hbm.at[c, pl.ds(k * CHUNK, CHUNK)], pbuf, sem)
        ck.wait()

        def phase_b(j, _, k=k):
            s = pl.ds(j * LANES, LANES)
            v = idx_v[s]
            val = plsc.load_gather(pbuf, [lax.bitwise_and(v, 65535)])
            hit = lax.eq(lax.shift_right_logical(v, 16), k)
            acc[s] = acc[s] + jnp.where(hit, val, 0.0)
            return 0

        lax.fori_loop(0, NGROUPS, phase_b, 0, unroll=4)

    pltpu.sync_copy(acc, out_hbm.at[c])


@jax.jit
def _octree_lookup(indices, cb0, cb1, cb2, cb3):
    ts = [cb.T for cb in (cb0, cb1, cb2, cb3)]
    mesh = plsc.VectorSubcoreMesh(core_axis_name="c", subcore_axis_name="s")
    f = functools.partial(
        pl.kernel,
        mesh=mesh,
        compiler_params=pltpu.CompilerParams(needs_layout_passes=False),
        out_type=jax.ShapeDtypeStruct((FEAT, BATCH), jnp.float32),
        scratch_types=[
            pltpu.VMEM((BATCH,), jnp.int32),
            pltpu.VMEM((BATCH,), jnp.float32),
            pltpu.VMEM((4096,), jnp.float32),
            pltpu.VMEM((16384,), jnp.float32),
            pltpu.VMEM((CHUNK,), jnp.float32),
            pltpu.SemaphoreType.DMA,
        ],
    )(_body)
    out_t = f(indices, *ts)
    return out_t.T


def kernel(indices, cb0, cb1, cb2, cb3):
    return _octree_lookup(indices.astype(jnp.int32), cb0, cb1, cb2, cb3)
